# baseline (speedup 1.0000x reference)
import jax
import jax.numpy as jnp
from jax import lax
from jax.experimental import pallas as pl

_N_CLASSES = 1000
_BLOCK_I = 64


def _onehot_body(idx_ref, out_ref):
    idx = idx_ref[...]
    classes = lax.broadcasted_iota(jnp.int32, out_ref.shape, 2)
    out_ref[...] = (idx[:, :, None] == classes).astype(jnp.float32)


def kernel(inputs):
    n, m = inputs.shape
    grid = (n // _BLOCK_I,)
    return pl.pallas_call(
        _onehot_body,
        grid=grid,
        in_specs=[pl.BlockSpec((_BLOCK_I, m), lambda i: (i, 0))],
        out_specs=pl.BlockSpec((_BLOCK_I, m, _N_CLASSES), lambda i: (i, 0, 0)),
        out_shape=jax.ShapeDtypeStruct((n, m, _N_CLASSES), jnp.float32),
    )(inputs)
